# Initial kernel scaffold; baseline (speedup 1.0000x reference)
import jax
import jax.numpy as jnp
from jax.experimental import pallas as pl


def kernel(x, edge_index, edge_attr, Wq0, bq0, Wk0, bk0, Wv0, bv0, Ws0, bs0, Wq1, bq1, Wk1, bk1, Wv1, bv1, Ws1, bs1, linW, linb):
    raise NotImplementedError("write your pallas kernel here")


# speedup vs baseline: 28.0306x; 28.0306x over previous
import functools
import math

import jax
import jax.numpy as jnp
from jax import lax
from jax.experimental import pallas as pl
from jax.experimental.pallas import tpu as pltpu
from jax.experimental.pallas import tpu_sc as plsc

_F32 = jnp.float32
_LANES = 128
_ACCW = 40


def _qkvs(xa, Wcat, bcat, bn):
    n, f = xa.shape

    def body(x_ref, w_ref, b_ref, tq_ref, tkv_ref, ts_ref):
        r = jnp.dot(x_ref[...], w_ref[...], preferred_element_type=_F32)
        r = r + b_ref[...]
        tq_ref[...] = r[:, 0:32]
        tkv_ref[...] = r[:, 32:96]
        ts_ref[...] = r[:, 96:128]

    return pl.pallas_call(
        body,
        grid=(n // bn,),
        in_specs=[
            pl.BlockSpec((bn, f), lambda i: (i, 0)),
            pl.BlockSpec((f, 128), lambda i: (0, 0)),
            pl.BlockSpec((1, 128), lambda i: (0, 0)),
        ],
        out_specs=[
            pl.BlockSpec((bn, 32), lambda i: (i, 0)),
            pl.BlockSpec((bn, 64), lambda i: (i, 0)),
            pl.BlockSpec((bn, 32), lambda i: (i, 0)),
        ],
        out_shape=[
            jax.ShapeDtypeStruct((n, 32), _F32),
            jax.ShapeDtypeStruct((n, 64), _F32),
            jax.ShapeDtypeStruct((n, 32), _F32),
        ],
    )(xa, Wcat, bcat)


def _alpha_of(qd, kv, scale):
    s = qd * kv[:, 0:32]
    a0 = jnp.sum(s[:, 0:16], axis=1, keepdims=True)
    a1 = jnp.sum(s[:, 16:32], axis=1, keepdims=True)
    return jnp.concatenate([a0, a1], axis=1) * scale


def _gmax(Qd, KVs, scale, be):
    e_pad = Qd.shape[0]

    def body(qd_ref, kv_ref, gm_ref):
        i = pl.program_id(0)
        alpha = _alpha_of(qd_ref[...], kv_ref[...], scale)
        bm = jnp.max(alpha, axis=0)[None, :]

        @pl.when(i == 0)
        def _():
            gm_ref[...] = jnp.full((1, 2), -3e38, _F32)

        gm_ref[...] = jnp.maximum(gm_ref[...], bm)

    return pl.pallas_call(
        body,
        grid=(e_pad // be,),
        in_specs=[
            pl.BlockSpec((be, 32), lambda i: (i, 0)),
            pl.BlockSpec((be, 64), lambda i: (i, 0)),
        ],
        out_specs=pl.BlockSpec((1, 2), lambda i: (0, 0)),
        out_shape=jax.ShapeDtypeStruct((1, 2), _F32),
    )(Qd, KVs)


def _msg(Qd, KVs, gm, scale, n_edges, be):
    e_pad = Qd.shape[0]

    def body(qd_ref, kv_ref, gm_ref, msg_ref):
        i = pl.program_id(0)
        kv = kv_ref[...]
        alpha = _alpha_of(qd_ref[...], kv, scale)
        ex = jnp.exp(alpha - gm_ref[...])
        row = i * be + lax.broadcasted_iota(jnp.int32, (be, 1), 0)
        ex = jnp.where(row < n_edges, ex, 0.0)
        vs = kv[:, 32:64]
        m32 = jnp.concatenate(
            [vs[:, 0:16] * ex[:, 0:1], vs[:, 16:32] * ex[:, 1:2]], axis=1)
        msg_ref[...] = jnp.concatenate(
            [m32, ex, jnp.zeros((be, 6), _F32)], axis=1)

    return pl.pallas_call(
        body,
        grid=(e_pad // be,),
        in_specs=[
            pl.BlockSpec((be, 32), lambda i: (i, 0)),
            pl.BlockSpec((be, 64), lambda i: (i, 0)),
            pl.BlockSpec((1, 2), lambda i: (0, 0)),
        ],
        out_specs=pl.BlockSpec((be, _ACCW), lambda i: (i, 0)),
        out_shape=jax.ShapeDtypeStruct((e_pad, _ACCW), _F32),
    )(Qd, KVs, gm)


def _finalize(acc2, Ts, W2, b2, do_elu, split, bn):
    n = Ts.shape[0]
    w = W2.shape[1]

    def body(a_ref, ts_ref, w_ref, b_ref, *o_refs):
        acc = a_ref[0] + a_ref[1]
        d0 = acc[:, 32:33] + 1e-16
        d1 = acc[:, 33:34] + 1e-16
        h = jnp.concatenate([acc[:, 0:16] / d0, acc[:, 16:32] / d1], axis=1)
        h = h + ts_ref[...]
        if do_elu:
            h = jnp.where(h > 0, h, jnp.exp(h) - 1.0)
        r = jnp.dot(h, w_ref[...], preferred_element_type=_F32) + b_ref[...]
        if split:
            o_refs[0][...] = r[:, 0:32]
            o_refs[1][...] = r[:, 32:96]
            o_refs[2][...] = r[:, 96:128]
        else:
            o_refs[0][...] = r

    if split:
        out_specs = [
            pl.BlockSpec((bn, 32), lambda i: (i, 0)),
            pl.BlockSpec((bn, 64), lambda i: (i, 0)),
            pl.BlockSpec((bn, 32), lambda i: (i, 0)),
        ]
        out_shape = [
            jax.ShapeDtypeStruct((n, 32), _F32),
            jax.ShapeDtypeStruct((n, 64), _F32),
            jax.ShapeDtypeStruct((n, 32), _F32),
        ]
    else:
        out_specs = pl.BlockSpec((bn, w), lambda i: (i, 0))
        out_shape = jax.ShapeDtypeStruct((n, w), _F32)

    return pl.pallas_call(
        body,
        grid=(n // bn,),
        in_specs=[
            pl.BlockSpec((2, bn, _ACCW), lambda i: (0, i, 0)),
            pl.BlockSpec((bn, 32), lambda i: (i, 0)),
            pl.BlockSpec((32, w), lambda i: (0, 0)),
            pl.BlockSpec((1, w), lambda i: (0, 0)),
        ],
        out_specs=out_specs,
        out_shape=out_shape,
    )(acc2, Ts, W2, b2)


def _sc_gather(Tq, Tkv, dst3, src3):
    nw, g, _ = dst3.shape
    e_pad = nw * g * _LANES
    ew = g * _LANES
    mesh = plsc.VectorSubcoreMesh(core_axis_name="c", subcore_axis_name="s")

    @functools.partial(
        pl.kernel,
        out_type=[
            jax.ShapeDtypeStruct((e_pad, 32), _F32),
            jax.ShapeDtypeStruct((e_pad, 64), _F32),
        ],
        mesh=mesh,
        scratch_types=[
            pltpu.VMEM((g, _LANES), jnp.int32),
            pltpu.VMEM((g, _LANES), jnp.int32),
            pltpu.VMEM((_LANES, 32), _F32),
            pltpu.VMEM((_LANES, 64), _F32),
            pltpu.SemaphoreType.DMA,
            pltpu.SemaphoreType.DMA,
        ],
        compiler_params=pltpu.CompilerParams(use_tc_tiling_on_sc=False),
    )
    def k(tq_hbm, tkv_hbm, dsti_hbm, srci_hbm, qd_hbm, kvs_hbm,
          dv, sv, qbuf, kvbuf, sem1, sem2):
        wid = lax.axis_index("s") * 2 + lax.axis_index("c")
        pltpu.sync_copy(dsti_hbm.at[wid], dv)
        pltpu.sync_copy(srci_hbm.at[wid], sv)

        def body(gi, carry):
            base = wid * ew + gi * _LANES
            cp1 = pltpu.async_copy(tq_hbm.at[dv.at[gi]], qbuf, sem1)
            cp2 = pltpu.async_copy(tkv_hbm.at[sv.at[gi]], kvbuf, sem2)
            cp1.wait()
            cp2.wait()
            pltpu.sync_copy(qbuf, qd_hbm.at[pl.ds(base, _LANES)])
            pltpu.sync_copy(kvbuf, kvs_hbm.at[pl.ds(base, _LANES)])
            return carry

        lax.fori_loop(0, g, body, 0)

    return k(Tq, Tkv, dst3, src3)


def _sc_scatter(msg, dst3, zeros_nw, n):
    nw, g, _ = dst3.shape
    ew = g * _LANES
    rows_per_tile = n // 16
    mesh = plsc.VectorSubcoreMesh(core_axis_name="c", subcore_axis_name="s")

    @functools.partial(
        pl.kernel,
        out_type=jax.ShapeDtypeStruct((2, n, _ACCW), _F32),
        mesh=mesh,
        scratch_types=[
            pltpu.VMEM((g, _LANES), jnp.int32),
            pltpu.VMEM((_LANES, _ACCW), _F32),
            pltpu.VMEM_SHARED((n, _ACCW), _F32),
        ],
        compiler_params=pltpu.CompilerParams(use_tc_tiling_on_sc=False),
    )
    def k(msg_hbm, dsti_hbm, z_hbm, out_hbm, dv, mbuf, acc_sh):
        c = lax.axis_index("c")
        s = lax.axis_index("s")
        wid = s * 2 + c
        rows = pl.ds(s * rows_per_tile, rows_per_tile)
        pltpu.sync_copy(z_hbm.at[rows], acc_sh.at[rows])
        pltpu.sync_copy(dsti_hbm.at[wid], dv)
        plsc.subcore_barrier()

        def body(gi, carry):
            base = wid * ew + gi * _LANES
            pltpu.sync_copy(msg_hbm.at[pl.ds(base, _LANES)], mbuf)
            pltpu.sync_copy(mbuf, acc_sh.at[dv.at[gi]], add=True)
            return carry

        lax.fori_loop(0, g, body, 0)
        plsc.subcore_barrier()
        pltpu.sync_copy(acc_sh.at[rows], out_hbm.at[c, rows])

    return k(msg, dst3, zeros_nw)


def kernel(x, edge_index, edge_attr,
           Wq0, bq0, Wk0, bk0, Wv0, bv0, Ws0, bs0,
           Wq1, bq1, Wk1, bk1, Wv1, bv1, Ws1, bs1,
           linW, linb):
    n, _ = x.shape
    e = edge_index.shape[1]
    hid = Wq0.shape[1]
    o = hid // 2
    scale = 1.0 / math.sqrt(o)

    nw = 32
    g = -(-e // (nw * _LANES))
    e_pad = nw * g * _LANES

    src = edge_index[0]
    dst = edge_index[1]
    pad = jnp.zeros((e_pad - e,), jnp.int32)
    src3 = jnp.concatenate([src, pad]).reshape(nw, g, _LANES)
    dst3 = jnp.concatenate([dst, pad]).reshape(nw, g, _LANES)

    Wc0 = jnp.concatenate([Wq0, Wk0, Wv0, Ws0], axis=1)
    bc0 = jnp.concatenate([bq0, bk0, bv0, bs0])[None, :]
    Wc1 = jnp.concatenate([Wq1, Wk1, Wv1, Ws1], axis=1)
    bc1 = jnp.concatenate([bq1, bk1, bv1, bs1])[None, :]
    z40 = jnp.zeros((n, _ACCW), _F32)

    bn = 1000
    be = 2048 if e_pad % 2048 == 0 else _LANES

    Tq, Tkv, Ts = _qkvs(x, Wc0, bc0, bn)
    Qd, KVs = _sc_gather(Tq, Tkv, dst3, src3)
    gm = _gmax(Qd, KVs, scale, be)
    m = _msg(Qd, KVs, gm, scale, e, be)
    acc2 = _sc_scatter(m, dst3, z40, n)
    Tq, Tkv, Ts1 = _finalize(acc2, Ts, Wc1, bc1, True, True, bn)
    Qd, KVs = _sc_gather(Tq, Tkv, dst3, src3)
    gm = _gmax(Qd, KVs, scale, be)
    m = _msg(Qd, KVs, gm, scale, e, be)
    acc2 = _sc_scatter(m, dst3, z40, n)
    out = _finalize(acc2, Ts1, linW, linb[None, :], False, False, bn)
    return out
